# baseline (speedup 1.0000x reference)
import functools

import jax
import jax.numpy as jnp
from jax import lax
from jax.experimental import pallas as pl
from jax.experimental.pallas import tpu as pltpu, tpu_sc as plsc

N = 4_194_304
GRID = 1024
GG = GRID * GRID
BOX = 500.0
FOV = 0.2
R_CENTER = 750.0
WIDTH = 100.0
A_CURRENT = 0.6

NC, NS, L = 2, 16, 16
NW = NC * NS
PER_W = N // NW
CH = 4096
NCHUNK = PER_W // CH
NGROUP = CH // L
SLICE = GG // NS
ZB = 8192
K = 256
KP = K + L


def _bf16_rne(v):
    u = lax.bitcast_convert_type(v, jnp.int32)
    bias = 0x7FFF + (lax.shift_right_logical(u, 16) & 1)
    u2 = (u + bias) & jnp.int32(-65536)
    return lax.bitcast_convert_type(u2, jnp.float32)


def _sc_paint(px_hbm, py_hbm, pz_hbm, vx_hbm, vy_hbm, vz_hbm, w_hbm,
              cst_hbm, out_hbm,
              pxb, pyb, pzb, vxb, vyb, vzb, wb, idxb, ctrb, cidx, cctr,
              cstb, zb, hist, insem):
    c = lax.axis_index("c")
    s = lax.axis_index("s")
    wid = c * NS + s

    zeros16 = jnp.zeros((L,), jnp.float32)

    def _zb_body(i, _):
        zb[pl.ds(i * L, L)] = zeros16
        return _

    lax.fori_loop(0, ZB // L, _zb_body, 0, unroll=4)
    for q in range(SLICE // ZB):
        pltpu.sync_copy(zb, hist.at[pl.ds(s * SLICE + q * ZB, ZB)])

    pltpu.sync_copy(cst_hbm, cstb)
    m00 = cstb[pl.ds(0, L)]
    m01 = cstb[pl.ds(16, L)]
    m02 = cstb[pl.ds(32, L)]
    m10 = cstb[pl.ds(48, L)]
    m11 = cstb[pl.ds(64, L)]
    m12 = cstb[pl.ds(80, L)]
    m20 = cstb[pl.ds(96, L)]
    m21 = cstb[pl.ds(112, L)]
    m22 = cstb[pl.ds(128, L)]
    o0 = cstb[pl.ds(144, L)]
    o1 = cstb[pl.ds(160, L)]
    o2 = cstb[pl.ds(176, L)]
    t_lo = cstb[pl.ds(192, L)]
    t_hi = cstb[pl.ds(208, L)]

    plsc.subcore_barrier()

    hbm_bufs = ((px_hbm, pxb), (py_hbm, pyb), (pz_hbm, pzb),
                (vx_hbm, vxb), (vy_hbm, vyb), (vz_hbm, vzb), (w_hbm, wb))

    def _fire(t):
        base = wid * PER_W + t * CH
        for src, dst in hbm_bufs:
            pltpu.async_copy(src.at[pl.ds(base, CH)], dst, insem)

    def _wait(t):
        base = wid * PER_W + t * CH
        for src, dst in hbm_bufs:
            pltpu.make_async_copy(src.at[pl.ds(base, CH)], dst, insem).wait()

    _fire(0)

    def _chunk(t, _):
        _wait(t)

        for q in range(KP // L):
            cctr[pl.ds(q * L, L)] = zeros16
            cidx[pl.ds(q * L, L)] = jnp.zeros((L,), jnp.int32)

        def _group(g, off):
            o = g * L
            px = pxb[pl.ds(o, L)]
            py = pyb[pl.ds(o, L)]
            pz = pzb[pl.ds(o, L)]
            vx = vxb[pl.ds(o, L)]
            vy = vyb[pl.ds(o, L)]
            vz = vzb[pl.ds(o, L)]
            w = wb[pl.ds(o, L)]

            dxp = _bf16_rne(px - o0)
            dyp = _bf16_rne(py - o1)
            dzp = _bf16_rne(pz - o2)
            vx = _bf16_rne(vx)
            vy = _bf16_rne(vy)
            vz = _bf16_rne(vz)
            x = m00 * dxp + m01 * dyp + m02 * dzp
            y = m10 * dxp + m11 * dyp + m12 * dzp
            z = m20 * dxp + m21 * dyp + m22 * dzp + 500.0
            rvx = m00 * vx + m01 * vy + m02 * vz
            rvy = m10 * vx + m11 * vy + m12 * vz
            rvz = m20 * vx + m21 * vy + m22 * vz

            d2 = jnp.maximum(x * x + y * y + z * z, 1e-12)
            ii = lax.bitcast_convert_type(d2, jnp.int32)
            ii = 0x5F3759DF - lax.shift_right_logical(ii, 1)
            yv = lax.bitcast_convert_type(ii, jnp.float32)
            for _i in range(3):
                yv = yv * (1.5 - 0.5 * d2 * yv * yv)
            dist = d2 * yv
            a_t = 1.0 / (1.0 + dist / 3000.0)
            drift = a_t - A_CURRENT
            xd = x + drift * rvx
            yd = y + drift * rvy
            zd = z + drift * rvz

            r2 = xd * xd + yd * yd + zd * zd
            in_shell = (r2 >= t_lo) & (r2 < t_hi) & (zd > 1e-3)
            zsafe = jnp.maximum(zd, 1e-3)
            sx = (xd / zsafe / FOV + 0.5) * 1024.0
            sy = (yd / zsafe / FOV + 0.5) * 1024.0
            valid = in_shell & (sx >= 0.0) & (sx < GRID) & (sy >= 0.0) & (sy < GRID)
            ix = jnp.clip(sx, 0.0, GRID - 1.0).astype(jnp.int32)
            iy = jnp.clip(sy, 0.0, GRID - 1.0).astype(jnp.int32)
            flat = lax.shift_left(iy, 10) + ix
            contrib = jnp.where(valid, w, 0.0)
            idxb[pl.ds(o, L)] = flat
            ctrb[pl.ds(o, L)] = contrib
            os_ = jnp.minimum(off, K)
            plsc.store_compressed(cidx.at[pl.ds(os_, L)], flat, mask=valid)
            plsc.store_compressed(cctr.at[pl.ds(os_, L)], contrib, mask=valid)
            cnt = plsc.all_reduce_population_count(valid)
            return off + jnp.max(cnt)

        tcnt = plsc.parallel_loop(0, NGROUP, 1, unroll=4,
                                  carry=jnp.int32(0))(_group)

        @pl.when(t < NCHUNK - 1)
        def _prefetch():
            _fire(t + 1)

        @pl.when(tcnt <= K)
        def _scatter_small():
            pltpu.sync_copy(cctr, hist.at[cidx], add=True)

        @pl.when(tcnt > K)
        def _scatter_full():
            pltpu.sync_copy(ctrb, hist.at[idxb], add=True)

        return _

    lax.fori_loop(0, NCHUNK, _chunk, 0)

    plsc.subcore_barrier()
    pltpu.sync_copy(hist.at[pl.ds(s * SLICE, SLICE)],
                    out_hbm.at[c, pl.ds(s * SLICE, SLICE)])


@jax.jit
def _paint(px, py, pz, vx, vy, vz, weights, consts):
    mesh = plsc.VectorSubcoreMesh(core_axis_name="c", subcore_axis_name="s",
                                  num_cores=NC, num_subcores=NS)
    comp = pltpu.VMEM((CH,), jnp.float32)
    return pl.kernel(
        _sc_paint,
        out_type=jax.ShapeDtypeStruct((NC, GG), jnp.float32),
        mesh=mesh,
        compiler_params=pltpu.CompilerParams(needs_layout_passes=False),
        scratch_types=[
            comp, comp, comp, comp, comp, comp,
            comp,
            pltpu.VMEM((CH,), jnp.int32),
            pltpu.VMEM((CH,), jnp.float32),
            pltpu.VMEM((KP,), jnp.int32),
            pltpu.VMEM((KP,), jnp.float32),
            pltpu.VMEM((14 * L,), jnp.float32),
            pltpu.VMEM((ZB,), jnp.float32),
            pltpu.VMEM_SHARED((GG,), jnp.float32),
            pltpu.SemaphoreType.DMA,
        ],
    )(px, py, pz, vx, vy, vz, weights, consts)


def _combine_body(p_ref, o_ref):
    s = p_ref[0] + p_ref[1]
    o_ref[...] = s.reshape(GRID // 16, GRID)


@jax.jit
def _combine(partials):
    return pl.pallas_call(
        _combine_body,
        out_shape=jax.ShapeDtypeStruct((GRID, GRID), jnp.float32),
        grid=(16,),
        in_specs=[pl.BlockSpec((NC, GG // 16), lambda i: (0, i))],
        out_specs=pl.BlockSpec((GRID // 16, GRID), lambda i: (i, 0)),
    )(partials)


def _sqrt_threshold(root):
    import numpy as np
    t0 = np.float32(root) * np.float32(root)
    ulp = np.spacing(t0)
    cands = jnp.float32(t0) + jnp.arange(-64, 65, dtype=jnp.float32) * jnp.float32(ulp)
    return jnp.min(jnp.where(jnp.sqrt(cands) >= root, cands, jnp.inf))


def kernel(positions, velocities, weights, rotation, observer):
    mf = rotation.astype(jnp.float32)
    consts = jnp.concatenate([
        mf.reshape(9), observer.astype(jnp.float32),
        _sqrt_threshold(R_CENTER - WIDTH / 2.0)[None],
        _sqrt_threshold(R_CENTER + WIDTH / 2.0)[None],
    ])
    consts_b = jnp.broadcast_to(consts[:, None], (14, L)).reshape(14 * L)
    partials = _paint(positions[:, 0], positions[:, 1], positions[:, 2],
                      velocities[:, 0], velocities[:, 1], velocities[:, 2],
                      weights, consts_b)
    return _combine(partials)
